# baseline (speedup 1.0000x reference)
import jax
import jax.numpy as jnp
from jax import lax
from jax.experimental import pallas as pl
from jax.experimental.pallas import tpu as pltpu
from jax.experimental.pallas import tpu_sc as plsc

NUM_EMB = 1000
D = 64
N = 819200

NC = 2
NS = 16
NW = NC * NS
LANES = 16

FEATS = 16
NQ = NW // (D // FEATS)
TOKENS_PER_Q = N // NQ
CHUNK = 2048
STEPS = TOKENS_PER_Q // CHUNK
GROUPS = CHUNK // LANES
NBUF = 3
OUTER = (STEPS + NBUF - 1) // NBUF


def _body(xt_hbm, idx_hbm, wt_hbm, out_hbm, w_v, acc, idxb,
          sem_x, sem_out):
    cid = lax.axis_index("c")
    sid = lax.axis_index("s")
    wid = sid * NC + cid
    fgroup = wid % (D // FEATS)
    shard = wid // (D // FEATS)
    f0 = fgroup * FEATS
    qbase = shard * TOKENS_PER_Q

    pltpu.sync_copy(wt_hbm.at[pl.ds(f0 * NUM_EMB, FEATS * NUM_EMB)], w_v)

    def x_in(k, b):
        return pltpu.make_async_copy(
            xt_hbm.at[pl.ds(f0, FEATS), pl.ds(qbase + k * CHUNK, CHUNK)],
            acc[b], sem_x[b])

    def i_in(k, b):
        return pltpu.make_async_copy(
            idx_hbm.at[pl.ds(qbase + k * CHUNK, CHUNK)], idxb[b], sem_x[b])

    def out_cp(k, b):
        return pltpu.make_async_copy(
            acc[b],
            out_hbm.at[pl.ds(f0, FEATS), pl.ds(qbase + k * CHUNK, CHUNK)],
            sem_out[b])

    for kk in (0, 1):
        x_in(kk, kk).start()
        i_in(kk, kk).start()

    def round_(g, carry):
        for j in range(NBUF):
            k = NBUF * g + j
            b = j

            @pl.when(k < STEPS)
            def _():
                x_in(k, b).wait()
                i_in(k, b).wait()

                @plsc.parallel_loop(0, GROUPS, 1, unroll=2)
                def add_group(gg):
                    ids = idxb[b][pl.ds(gg * LANES, LANES)]
                    for f in range(FEATS):
                        wv = plsc.load_gather(w_v, [ids + f * NUM_EMB])
                        plsc.addupdate(
                            acc[b].at[f, pl.ds(gg * LANES, LANES)], wv)

                out_cp(k, b).start()

                b2 = (j + 2) % NBUF

                @pl.when(k + 2 < STEPS)
                def _():
                    @pl.when(k >= 1)
                    def _():
                        out_cp(k - 1, b2).wait()
                    x_in(k + 2, b2).start()
                    i_in(k + 2, b2).start()

        return carry

    lax.fori_loop(0, OUTER, round_, 0)
    for k in range(STEPS - NBUF, STEPS):
        out_cp(k, k % NBUF).wait()


@jax.jit
def _run(xt, idx, wt):
    mesh = plsc.VectorSubcoreMesh(core_axis_name="c", subcore_axis_name="s")
    f = pl.kernel(
        _body,
        out_type=jax.ShapeDtypeStruct((D, N), jnp.float32),
        mesh=mesh,
        compiler_params=pltpu.CompilerParams(needs_layout_passes=False),
        scratch_types=[
            pltpu.VMEM((FEATS * NUM_EMB,), jnp.float32),
            [pltpu.VMEM((FEATS, CHUNK), jnp.float32)] * NBUF,
            [pltpu.VMEM((CHUNK,), jnp.int32)] * NBUF,
            [pltpu.SemaphoreType.DMA] * NBUF,
            [pltpu.SemaphoreType.DMA] * NBUF,
        ],
    )
    return f(xt, idx, wt)


def kernel(x, resids_positional_encoded, W):
    idx = resids_positional_encoded.astype(jnp.int32)
    wt = jnp.reshape(W.T, (-1,))
    return _run(x.T, idx, wt).T
